# Initial kernel scaffold; baseline (speedup 1.0000x reference)
import jax
import jax.numpy as jnp
from jax.experimental import pallas as pl


def kernel(bert_indices_add, wte, learned_embedding_1, learned_embedding_2):
    raise NotImplementedError("write your pallas kernel here")


# speedup vs baseline: 3.7752x; 3.7752x over previous
import functools

import jax
import jax.numpy as jnp
from jax import lax
from jax.experimental import pallas as pl
from jax.experimental.pallas import tpu as pltpu
from jax.experimental.pallas import tpu_sc as plsc

N_TOK = 10
_info = plsc.get_sparse_core_info()
_NC = _info.num_cores
_NS = _info.num_subcores
_NW = _NC * _NS


def _make_gather(nb, seq, d, t):
  mesh = plsc.VectorSubcoreMesh(core_axis_name="c", subcore_axis_name="s")
  iters = (nb + _NW - 1) // _NW
  g0 = 128
  g1 = seq - g0
  g1p = seq - N_TOK - g0

  @functools.partial(
      pl.kernel,
      mesh=mesh,
      out_type=jax.ShapeDtypeStruct((nb * seq, d), jnp.float32),
      scratch_types=[
          pltpu.VMEM((seq,), jnp.int32),
          pltpu.VMEM((N_TOK + seq, d), jnp.float32),
          pltpu.SemaphoreType.DMA,
      ],
  )
  def k(idx_hbm, table_hbm, learned_hbm, out_hbm, idx_v, rows_v, sem):
    wid = lax.axis_index("s") * _NC + lax.axis_index("c")

    def body(i, _):
      b = wid + i * _NW

      @pl.when(b < nb)
      def _():
        base = b * seq
        pltpu.sync_copy(idx_hbm.at[pl.ds(base, seq)], idx_v)

        @pl.when((b >= t) & (b < t + _NW))
        def _():
          pltpu.sync_copy(learned_hbm.at[pl.ds(0, 16)], rows_v.at[pl.ds(0, 16)])

        @pl.when((b >= 2 * t) & (b < 2 * t + _NW))
        def _():
          pltpu.sync_copy(
              learned_hbm.at[pl.ds(16, 16)], rows_v.at[pl.ds(0, 16)])

        @pl.when(b < t)
        def _():
          c0 = pltpu.async_copy(
              table_hbm.at[idx_v.at[pl.ds(0, g0)]],
              rows_v.at[pl.ds(N_TOK, g0)], sem)
          c1 = pltpu.async_copy(
              table_hbm.at[idx_v.at[pl.ds(g0, g1)]],
              rows_v.at[pl.ds(N_TOK + g0, g1)], sem)
          c0.wait()
          c1.wait()
          pltpu.sync_copy(
              rows_v.at[pl.ds(N_TOK, seq)], out_hbm.at[pl.ds(base, seq)])

        @pl.when(b >= t)
        def _():
          c0 = pltpu.async_copy(
              table_hbm.at[idx_v.at[pl.ds(0, g0)]],
              rows_v.at[pl.ds(N_TOK, g0)], sem)
          c1 = pltpu.async_copy(
              table_hbm.at[idx_v.at[pl.ds(g0, g1p)]],
              rows_v.at[pl.ds(N_TOK + g0, g1p)], sem)
          c0.wait()
          c1.wait()
          pltpu.sync_copy(
              rows_v.at[pl.ds(0, seq)], out_hbm.at[pl.ds(base, seq)])
      return 0

    lax.fori_loop(0, iters, body, 0)

  return k


def kernel(bert_indices_add, wte, learned_embedding_1, learned_embedding_2):
  B, S = bert_indices_add.shape
  t = B // 3
  nb = 3 * t
  n_tok = learned_embedding_1.shape[0]
  d = wte.shape[1]
  idx = bert_indices_add.astype(jnp.int32)
  shifted = jnp.pad(idx[t:nb, : S - n_tok], ((0, 0), (0, n_tok)))
  flat_idx = jnp.concatenate([idx[:t], shifted], axis=0).reshape(-1)
  pad = jnp.zeros((16 - n_tok, d), jnp.float32)
  learned = jnp.concatenate(
      [learned_embedding_1, pad, learned_embedding_2, pad], axis=0)
  out = _make_gather(nb, S, d, t)(flat_idx, wte, learned)
  return out.reshape(nb, S, d)
